# baseline (speedup 1.0000x reference)
import functools

import jax
import jax.numpy as jnp
from jax import lax
from jax.experimental import pallas as pl
from jax.experimental.pallas import tpu as pltpu
from jax.experimental.pallas import tpu_sc as plsc

B = 4096
D0, D1, D2, DF = 32, 64, 64, 64
COMB = D0 + D1 + D2 + DF
NC, NS, L = 2, 16, 16
NW = NC * NS
BPW = B // NW
NCHUNK = COMB // L
STRIDE = BPW + 1


def _build():
    mesh = plsc.VectorSubcoreMesh(core_axis_name="c", subcore_axis_name="s")

    @functools.partial(
        pl.kernel,
        mesh=mesh,
        out_type=jax.ShapeDtypeStruct((B,), jnp.float32),
        compiler_params=pltpu.CompilerParams(
            needs_layout_passes=False, use_tc_tiling_on_sc=False),
        scratch_types=[
            pltpu.VMEM((BPW,), jnp.int32),
            pltpu.VMEM((BPW,), jnp.int32),
            pltpu.VMEM((BPW,), jnp.int32),
            pltpu.VMEM((BPW, D0), jnp.float32),
            pltpu.VMEM((BPW, D1), jnp.float32),
            pltpu.VMEM((BPW, D2), jnp.float32),
            pltpu.VMEM((BPW, DF), jnp.float32),
            pltpu.VMEM((COMB,), jnp.float32),
            pltpu.VMEM((L,), jnp.float32),
            pltpu.VMEM((BPW * L,), jnp.float32),
            pltpu.VMEM((BPW,), jnp.float32),
            pltpu.SemaphoreType.DMA,
        ],
    )
    def sc_kernel(id0_h, id1_h, id2_h, feat_h, t0_h, t1_h, t2_h, w_h, b_h,
                  out_h, idx0, idx1, idx2, r0, r1, r2, fv, wv, bv, accT,
                  outv, sem):
        wid = lax.axis_index("s") * NC + lax.axis_index("c")
        base = wid * BPW
        pltpu.sync_copy(id0_h.at[pl.ds(base, BPW)], idx0)
        pltpu.sync_copy(id1_h.at[pl.ds(base, BPW)], idx1)
        pltpu.sync_copy(id2_h.at[pl.ds(base, BPW)], idx2)
        cp0 = pltpu.async_copy(t0_h.at[idx0], r0, sem)
        cp1 = pltpu.async_copy(t1_h.at[idx1], r1, sem)
        cp2 = pltpu.async_copy(t2_h.at[idx2], r2, sem)
        pltpu.sync_copy(feat_h.at[pl.ds(base, BPW)], fv)
        pltpu.sync_copy(w_h, wv)
        pltpu.sync_copy(b_h, bv)
        cp0.wait()
        cp1.wait()
        cp2.wait()

        wch = [wv[pl.ds(c * L, L)] for c in range(NCHUNK)]
        iot = lax.iota(jnp.int32, L)
        row_base = iot * L

        def body(i, carry):
            acc = r0[i, pl.ds(0, L)] * wch[0]
            acc = acc + r0[i, pl.ds(L, L)] * wch[1]
            o = 2
            for c in range(D1 // L):
                acc = acc + r1[i, pl.ds(c * L, L)] * wch[o + c]
            o += D1 // L
            for c in range(D2 // L):
                acc = acc + r2[i, pl.ds(c * L, L)] * wch[o + c]
            o += D2 // L
            for c in range(DF // L):
                acc = acc + fv[i, pl.ds(c * L, L)] * wch[o + c]
            accT[pl.ds(i * L, L)] = acc
            return carry

        lax.fori_loop(0, BPW, body, 0)

        for g in range(BPW // L):
            gbase = row_base + g * (L * L)
            s = plsc.load_gather(accT, [gbase])
            for k in range(1, L):
                s = s + plsc.load_gather(accT, [gbase + k])
            outv[pl.ds(g * L, L)] = s + bv[...]
        pltpu.sync_copy(outv, out_h.at[pl.ds(base, BPW)])

    return sc_kernel


_SC_KERNEL = _build()


def kernel(hierarchy_ids_level0, hierarchy_ids_level1, hierarchy_ids_level2,
           features, emb_level0, emb_level1, emb_level2, W, b):
    id0 = hierarchy_ids_level0.astype(jnp.int32)
    id1 = hierarchy_ids_level1.astype(jnp.int32)
    id2 = hierarchy_ids_level2.astype(jnp.int32)
    w_flat = W.reshape(-1).astype(jnp.float32)
    b_vec = jnp.broadcast_to(b.astype(jnp.float32), (L,))
    return _SC_KERNEL(id0, id1, id2, features, emb_level0, emb_level1,
                      emb_level2, w_flat, b_vec)


# speedup vs baseline: 5.2426x; 5.2426x over previous
import functools

import jax
import jax.numpy as jnp
from jax import lax
from jax.experimental import pallas as pl
from jax.experimental.pallas import tpu as pltpu
from jax.experimental.pallas import tpu_sc as plsc

B = 4096
D0, D1, D2, DF = 32, 64, 64, 64
V0, V1, V2 = 1000, 100000, 1000000
NC, NS, L = 2, 16, 16
NW = NC * NS
BPW = B // NW


def _matvec(d, v, blk):
    grid = (v + blk - 1) // blk

    def body(w_ref, t_ref, o_ref):
        o_ref[...] = jnp.dot(
            w_ref[...], t_ref[...], preferred_element_type=jnp.float32
        )[0]

    return pl.pallas_call(
        body,
        grid=(grid,),
        in_specs=[
            pl.BlockSpec((1, d), lambda i: (0, 0)),
            pl.BlockSpec((d, blk), lambda i: (0, i)),
        ],
        out_specs=pl.BlockSpec((blk,), lambda i: (i,)),
        out_shape=jax.ShapeDtypeStruct((v,), jnp.float32),
    )


_MV0 = _matvec(D0, V0, 1024)
_MV1 = _matvec(D1, V1, 16384)
_MV2 = _matvec(D2, V2, 16384)
_MVF = _matvec(DF, B, 4096)


def _build_gather():
    mesh = plsc.VectorSubcoreMesh(core_axis_name="c", subcore_axis_name="s")

    @functools.partial(
        pl.kernel,
        mesh=mesh,
        out_type=jax.ShapeDtypeStruct((B,), jnp.float32),
        compiler_params=pltpu.CompilerParams(
            needs_layout_passes=False, use_tc_tiling_on_sc=False),
        scratch_types=[
            pltpu.VMEM((BPW,), jnp.int32),
            pltpu.VMEM((BPW,), jnp.int32),
            pltpu.VMEM((BPW,), jnp.int32),
            pltpu.VMEM((BPW,), jnp.float32),
            pltpu.VMEM((BPW,), jnp.float32),
            pltpu.VMEM((BPW,), jnp.float32),
            pltpu.VMEM((BPW,), jnp.float32),
            pltpu.VMEM((L,), jnp.float32),
            pltpu.VMEM((BPW,), jnp.float32),
            pltpu.SemaphoreType.DMA,
        ],
    )
    def sc_kernel(id0_h, id1_h, id2_h, p0_h, p1_h, p2_h, fd_h, b_h, out_h,
                  idx0, idx1, idx2, g0, g1, g2, fdv, bv, outv, sem):
        wid = lax.axis_index("s") * NC + lax.axis_index("c")
        base = wid * BPW
        pltpu.sync_copy(id0_h.at[pl.ds(base, BPW)], idx0)
        pltpu.sync_copy(id1_h.at[pl.ds(base, BPW)], idx1)
        pltpu.sync_copy(id2_h.at[pl.ds(base, BPW)], idx2)
        cp0 = pltpu.async_copy(p0_h.at[idx0], g0, sem)
        cp1 = pltpu.async_copy(p1_h.at[idx1], g1, sem)
        cp2 = pltpu.async_copy(p2_h.at[idx2], g2, sem)
        pltpu.sync_copy(fd_h.at[pl.ds(base, BPW)], fdv)
        pltpu.sync_copy(b_h, bv)
        cp0.wait()
        cp1.wait()
        cp2.wait()
        bvec = bv[...]
        for c in range(BPW // L):
            sl = pl.ds(c * L, L)
            outv[sl] = g0[sl] + g1[sl] + g2[sl] + fdv[sl] + bvec
        pltpu.sync_copy(outv, out_h.at[pl.ds(base, BPW)])

    return sc_kernel


_SC_GATHER = _build_gather()


def kernel(hierarchy_ids_level0, hierarchy_ids_level1, hierarchy_ids_level2,
           features, emb_level0, emb_level1, emb_level2, W, b):
    id0 = hierarchy_ids_level0.astype(jnp.int32)
    id1 = hierarchy_ids_level1.astype(jnp.int32)
    id2 = hierarchy_ids_level2.astype(jnp.int32)
    w0 = W[:, :D0]
    w1 = W[:, D0:D0 + D1]
    w2 = W[:, D0 + D1:D0 + D1 + D2]
    wf = W[:, D0 + D1 + D2:]
    p0 = _MV0(w0, emb_level0.T)
    p1 = _MV1(w1, emb_level1.T)
    p2 = _MV2(w2, emb_level2.T)
    fd = _MVF(wf, features.T)
    b_vec = jnp.broadcast_to(b.astype(jnp.float32), (L,))
    return _SC_GATHER(id0, id1, id2, p0, p1, p2, fd, b_vec)


# speedup vs baseline: 5.7340x; 1.0937x over previous
import functools

import jax
import jax.numpy as jnp
from jax import lax
from jax.experimental import pallas as pl
from jax.experimental.pallas import tpu as pltpu
from jax.experimental.pallas import tpu_sc as plsc

B = 4096
D0, D1, D2, DF = 32, 64, 64, 64
V0, V1, V2 = 1000, 100000, 1000000
NC, NS, L = 2, 16, 16
NW = NC * NS
BPW = B // NW


def _matvec(d, v, blk):
    grid = (v + blk - 1) // blk

    def body(w_ref, t_ref, o_ref):
        o_ref[...] = jnp.dot(
            w_ref[...], t_ref[...], preferred_element_type=jnp.float32
        )[0]

    return pl.pallas_call(
        body,
        grid=(grid,),
        in_specs=[
            pl.BlockSpec((1, d), lambda i: (0, 0)),
            pl.BlockSpec((d, blk), lambda i: (0, i)),
        ],
        out_specs=pl.BlockSpec((blk,), lambda i: (i,)),
        out_shape=jax.ShapeDtypeStruct((v,), jnp.float32),
    )


_MV0 = _matvec(D0, V0, 1024)
_MV1 = _matvec(D1, V1, 32768)
_MV2 = _matvec(D2, V2, 32768)
_MVF = _matvec(DF, B, 4096)


def _build_gather():
    mesh = plsc.VectorSubcoreMesh(core_axis_name="c", subcore_axis_name="s")

    @functools.partial(
        pl.kernel,
        mesh=mesh,
        out_type=jax.ShapeDtypeStruct((B,), jnp.float32),
        compiler_params=pltpu.CompilerParams(
            needs_layout_passes=False, use_tc_tiling_on_sc=False),
        scratch_types=[
            pltpu.VMEM((BPW,), jnp.int32),
            pltpu.VMEM((BPW,), jnp.int32),
            pltpu.VMEM((BPW,), jnp.int32),
            pltpu.VMEM((BPW,), jnp.float32),
            pltpu.VMEM((BPW,), jnp.float32),
            pltpu.VMEM((BPW,), jnp.float32),
            pltpu.VMEM((BPW,), jnp.float32),
            pltpu.VMEM((L,), jnp.float32),
            pltpu.VMEM((BPW,), jnp.float32),
            pltpu.SemaphoreType.DMA,
        ],
    )
    def sc_kernel(id0_h, id1_h, id2_h, p0_h, p1_h, p2_h, fd_h, b_h, out_h,
                  idx0, idx1, idx2, g0, g1, g2, fdv, bv, outv, sem):
        wid = lax.axis_index("s") * NC + lax.axis_index("c")
        base = wid * BPW
        pltpu.sync_copy(id0_h.at[pl.ds(base, BPW)], idx0)
        pltpu.sync_copy(id1_h.at[pl.ds(base, BPW)], idx1)
        pltpu.sync_copy(id2_h.at[pl.ds(base, BPW)], idx2)
        cp0 = pltpu.async_copy(p0_h.at[idx0], g0, sem)
        cp1 = pltpu.async_copy(p1_h.at[idx1], g1, sem)
        cp2 = pltpu.async_copy(p2_h.at[idx2], g2, sem)
        pltpu.sync_copy(fd_h.at[pl.ds(base, BPW)], fdv)
        pltpu.sync_copy(b_h, bv)
        cp0.wait()
        cp1.wait()
        cp2.wait()
        bvec = bv[...]
        for c in range(BPW // L):
            sl = pl.ds(c * L, L)
            outv[sl] = g0[sl] + g1[sl] + g2[sl] + fdv[sl] + bvec
        pltpu.sync_copy(outv, out_h.at[pl.ds(base, BPW)])

    return sc_kernel


_SC_GATHER = _build_gather()


def kernel(hierarchy_ids_level0, hierarchy_ids_level1, hierarchy_ids_level2,
           features, emb_level0, emb_level1, emb_level2, W, b):
    id0 = hierarchy_ids_level0.astype(jnp.int32)
    id1 = hierarchy_ids_level1.astype(jnp.int32)
    id2 = hierarchy_ids_level2.astype(jnp.int32)
    w0 = W[:, :D0]
    w1 = W[:, D0:D0 + D1]
    w2 = W[:, D0 + D1:D0 + D1 + D2]
    wf = W[:, D0 + D1 + D2:]
    p0 = _MV0(w0, emb_level0.T)
    p1 = _MV1(w1, emb_level1.T)
    p2 = _MV2(w2, emb_level2.T)
    fd = _MVF(wf, features.T)
    b_vec = jnp.broadcast_to(b.astype(jnp.float32), (L,))
    return _SC_GATHER(id0, id1, id2, p0, p1, p2, fd, b_vec)
